# baseline (speedup 1.0000x reference)
import functools

import jax
import jax.numpy as jnp
import numpy as np
from jax import lax
from jax.experimental import pallas as pl
from jax.experimental.pallas import tpu as pltpu
from jax.experimental.pallas import tpu_sc as plsc

N = 50000
E = 800000
C = 32
H = 4
HD = 8
NRBF = 16
NT = 15

NC = 2
NS = 16
NW = NC * NS
PER_W = E // NW
CB = 128
NCH = PER_W // CB
TB = PER_W - NCH * CB
SW = 36
CBS = 4000

BN = 2048
BE = 2048

_INV_SQRT_HD = float(1.0 / np.sqrt(HD))


def _sc_gather(tsrc, src_idx, tdst, dst_idx, w1, w2):
    mesh = plsc.VectorSubcoreMesh(core_axis_name="c", subcore_axis_name="s")

    @functools.partial(
        pl.kernel,
        mesh=mesh,
        compiler_params=pltpu.CompilerParams(use_tc_tiling_on_sc=False),
        out_type=[
            jax.ShapeDtypeStruct((E, w1), jnp.float32),
            jax.ShapeDtypeStruct((E, w2), jnp.float32),
        ],
        scratch_types=[
            pltpu.VMEM((CB,), jnp.int32), pltpu.VMEM((CB,), jnp.int32),
            pltpu.VMEM((CB, w1), jnp.float32), pltpu.VMEM((CB, w2), jnp.float32),
            pltpu.VMEM((CB,), jnp.int32), pltpu.VMEM((CB,), jnp.int32),
            pltpu.VMEM((CB, w1), jnp.float32), pltpu.VMEM((CB, w2), jnp.float32),
            pltpu.SemaphoreType.DMA, pltpu.SemaphoreType.DMA,
        ],
    )
    def gather2_k(tsrc_hbm, src_hbm, tdst_hbm, dst_hbm, g1_hbm, g2_hbm,
                  i1a, i2a, r1a, r2a, i1b, i2b, r1b, r2b, sema, semb):
        wid = lax.axis_index("s") * NC + lax.axis_index("c")
        base_w = wid * PER_W

        def stage(base, i1, i2, r1, r2, sem):
            pltpu.sync_copy(src_hbm.at[pl.ds(base, CB)], i1)
            pltpu.sync_copy(dst_hbm.at[pl.ds(base, CB)], i2)
            pltpu.async_copy(tsrc_hbm.at[i1], r1, sem)
            pltpu.async_copy(tdst_hbm.at[i2], r2, sem)

        def finish(base, i1, i2, r1, r2, sem):
            pltpu.make_async_copy(tsrc_hbm.at[i1], r1, sem).wait()
            pltpu.make_async_copy(tdst_hbm.at[i2], r2, sem).wait()
            pltpu.sync_copy(r1, g1_hbm.at[pl.ds(base, CB)])
            pltpu.sync_copy(r2, g2_hbm.at[pl.ds(base, CB)])

        def body(j, carry):
            b0 = base_w + (2 * j) * CB
            b1 = b0 + CB
            stage(b0, i1a, i2a, r1a, r2a, sema)
            stage(b1, i1b, i2b, r1b, r2b, semb)
            finish(b0, i1a, i2a, r1a, r2a, sema)
            finish(b1, i1b, i2b, r1b, r2b, semb)
            return carry

        lax.fori_loop(0, NCH // 2, body, 0)

        def chunk(base, nb, i1, r1, i2, r2):
            pltpu.sync_copy(src_hbm.at[pl.ds(base, nb)], i1)
            pltpu.sync_copy(dst_hbm.at[pl.ds(base, nb)], i2)
            pltpu.async_copy(tsrc_hbm.at[i1], r1, sema)
            pltpu.async_copy(tdst_hbm.at[i2], r2, semb)
            pltpu.make_async_copy(tsrc_hbm.at[i1], r1, sema).wait()
            pltpu.make_async_copy(tdst_hbm.at[i2], r2, semb).wait()
            pltpu.sync_copy(r1, g1_hbm.at[pl.ds(base, nb)])
            pltpu.sync_copy(r2, g2_hbm.at[pl.ds(base, nb)])

        if NCH % 2 == 1:
            chunk(base_w + (NCH - 1) * CB, CB, i1a, r1a, i2a, r2a)
        chunk(base_w + NCH * CB, TB, i1a.at[pl.ds(0, TB)],
              r1a.at[pl.ds(0, TB)], i2a.at[pl.ds(0, TB)],
              r2a.at[pl.ds(0, TB)])

    return gather2_k(tsrc, src_idx, tdst, dst_idx)


def _sc_scatter(sT, dst_idx):
    mesh = plsc.VectorSubcoreMesh(core_axis_name="c", subcore_axis_name="s")
    npair = SW // 2

    @functools.partial(
        pl.kernel,
        mesh=mesh,
        compiler_params=pltpu.CompilerParams(use_tc_tiling_on_sc=False,
                                             needs_layout_passes=False),
        out_type=jax.ShapeDtypeStruct((SW, N), jnp.float32),
        scratch_types=[
            pltpu.VMEM((CBS,), jnp.int32), pltpu.VMEM((CBS,), jnp.float32),
            pltpu.VMEM((CBS,), jnp.float32),
            pltpu.VMEM((CBS,), jnp.int32), pltpu.VMEM((CBS,), jnp.float32),
            pltpu.VMEM((CBS,), jnp.float32),
            pltpu.VMEM((2 * N,), jnp.float32),
            pltpu.SemaphoreType.DMA, pltpu.SemaphoreType.DMA,
        ],
    )
    def scatter_k(s_hbm, dst_hbm, out_hbm, ia, va0, va1, ib, vb0, vb1, acc,
                  sema, semb):
        wid = lax.axis_index("s") * NC + lax.axis_index("c")

        @pl.when(wid < npair)
        def _():
            ch0 = wid * 2
            zv = jnp.zeros((16,), jnp.float32)
            nch2 = E // CBS // 2

            def zb(j, carry):
                acc[pl.ds(j * 16, 16)] = zv
                return carry

            lax.fori_loop(0, (2 * N) // 16, zb, 0)

            def start(i, iv, v0, v1, sem):
                base = i * CBS
                pltpu.async_copy(dst_hbm.at[pl.ds(base, CBS)], iv, sem)
                pltpu.async_copy(s_hbm.at[ch0, pl.ds(base, CBS)], v0, sem)
                pltpu.async_copy(s_hbm.at[ch0 + 1, pl.ds(base, CBS)], v1, sem)

            def drain(i, iv, v0, v1, sem):
                base = i * CBS
                pltpu.make_async_copy(dst_hbm.at[pl.ds(base, CBS)], iv, sem).wait()
                pltpu.make_async_copy(s_hbm.at[ch0, pl.ds(base, CBS)], v0, sem).wait()
                pltpu.make_async_copy(s_hbm.at[ch0 + 1, pl.ds(base, CBS)], v1, sem).wait()

            def proc(iv, v0, v1):
                def inner(k, c2):
                    idx16 = iv[pl.ds(k * 16, 16)]
                    plsc.addupdate_scatter(acc, [idx16], v0[pl.ds(k * 16, 16)])
                    plsc.addupdate_scatter(acc, [idx16 + N], v1[pl.ds(k * 16, 16)])
                    return c2

                lax.fori_loop(0, CBS // 16, inner, 0, unroll=4)

            start(0, ia, va0, va1, sema)

            def body(jj, carry):
                i0 = 2 * jj
                start(i0 + 1, ib, vb0, vb1, semb)
                drain(i0, ia, va0, va1, sema)
                proc(ia, va0, va1)

                @pl.when(i0 + 2 < 2 * nch2)
                def _start_next():
                    start(i0 + 2, ia, va0, va1, sema)

                drain(i0 + 1, ib, vb0, vb1, semb)
                proc(ib, vb0, vb1)
                return carry

            lax.fori_loop(0, nch2, body, 0)
            pltpu.sync_copy(acc.at[pl.ds(0, N)], out_hbm.at[ch0])
            pltpu.sync_copy(acc.at[pl.ds(N, N)], out_hbm.at[ch0 + 1])

    return scatter_k(sT, dst_idx)


def _full(shape):
    return pl.BlockSpec(shape, lambda i: tuple(0 for _ in shape))


def _rows(block_shape):
    return pl.BlockSpec(block_shape, lambda i: (i,) + tuple(0 for _ in block_shape[1:]))


def _prep1_body(h_ref, crp_ref, wq_ref, wkh_ref, wvh_ref, tsrc_ref, tdst_ref):
    h = h_ref[...]
    crp = crp_ref[...]
    q = jnp.dot(h, wq_ref[...], preferred_element_type=jnp.float32)
    ak = jnp.dot(h, wkh_ref[...], preferred_element_type=jnp.float32)
    av = jnp.dot(h, wvh_ref[...], preferred_element_type=jnp.float32)
    tsrc_ref[...] = jnp.concatenate([ak, av, crp], axis=1)
    tdst_ref[...] = jnp.concatenate([q, crp], axis=1)


def _edge1_body(g1_ref, g2_ref, e_ref, wk_ref, wv_ref, mh_ref, m4_ref,
                s_ref, rb_ref):
    g1 = g1_ref[...]
    g2 = g2_ref[...]
    relp = g2[:, C:C + 16] - g1[:, 2 * C:2 * C + 16]
    d = jnp.sqrt(jnp.sum(relp * relp, axis=1, keepdims=True) + 1e-8)
    centers = lax.broadcasted_iota(jnp.int32, (1, NRBF), 1).astype(
        jnp.float32) * (4.0 / (NRBF - 1))
    rb = jnp.exp(-((d - centers) ** 2) * 2.0)
    kin = jnp.concatenate([e_ref[...], rb], axis=1)
    k = g1[:, 0:C] + jnp.dot(kin, wk_ref[...], preferred_element_type=jnp.float32)
    v = g1[:, C:2 * C] + jnp.dot(kin, wv_ref[...], preferred_element_type=jnp.float32)
    prod = g2[:, 0:C] * k
    l32 = jnp.dot(prod, mh_ref[...], preferred_element_type=jnp.float32) * _INV_SQRT_HD
    l4 = jnp.dot(prod, m4_ref[...], preferred_element_type=jnp.float32) * _INV_SQRT_HD
    w32 = jnp.exp(l32)
    w4 = jnp.exp(l4)
    s_ref[...] = jnp.concatenate([w32 * v, w4], axis=1)
    rb_ref[...] = rb


def _edge2_body(g1_ref, g2_ref, e_ref, rb_ref, wk_ref, wv_ref, mh_ref, m4_ref,
                s_ref):
    g1 = g1_ref[...]
    g2 = g2_ref[...]
    kin = jnp.concatenate([e_ref[...], rb_ref[...]], axis=1)
    k = g1[:, 0:C] + jnp.dot(kin, wk_ref[...], preferred_element_type=jnp.float32)
    v = g1[:, C:2 * C] + jnp.dot(kin, wv_ref[...], preferred_element_type=jnp.float32)
    prod = g2[:, 0:C] * k
    l32 = jnp.dot(prod, mh_ref[...], preferred_element_type=jnp.float32) * _INV_SQRT_HD
    l4 = jnp.dot(prod, m4_ref[...], preferred_element_type=jnp.float32) * _INV_SQRT_HD
    w32 = jnp.exp(l32)
    w4 = jnp.exp(l4)
    s_ref[...] = jnp.concatenate([w32 * v, w4], axis=1)


def _agg_h(pT, h, rep, wo):
    numerT = pT[0:C, :]
    denT = pT[C:C + H, :]
    den32T = lax.dot_general(rep, denT, (((0,), (0,)), ((), ())),
                             preferred_element_type=jnp.float32)
    aggT = numerT / (den32T + 1e-9)
    upd = lax.dot_general(aggT, wo, (((0,), (0,)), ((), ())),
                          preferred_element_type=jnp.float32)
    return h + jnp.maximum(upd, 0.0)


def _upd1_body(pT_ref, h_ref, rep_ref, wo_ref, wq_ref, wkh_ref, wvh_ref,
               h2_ref, tsrc_ref, tdst_ref):
    h2 = _agg_h(pT_ref[...], h_ref[...], rep_ref[...], wo_ref[...])
    h2_ref[...] = h2
    ak = jnp.dot(h2, wkh_ref[...], preferred_element_type=jnp.float32)
    av = jnp.dot(h2, wvh_ref[...], preferred_element_type=jnp.float32)
    tsrc_ref[...] = jnp.concatenate([ak, av], axis=1)
    tdst_ref[...] = jnp.dot(h2, wq_ref[...], preferred_element_type=jnp.float32)


def _upd2_body(pT_ref, h_ref, rep_ref, wo_ref, wout_ref, wct_ref,
               hs0_ref, cs_ref):
    h2 = _agg_h(pT_ref[...], h_ref[...], rep_ref[...], wo_ref[...])
    hs0 = jnp.dot(h2, wout_ref[...], preferred_element_type=jnp.float32)
    hs0_ref[...] = hs0
    cs_ref[...] = jnp.dot(hs0, wct_ref[...], preferred_element_type=jnp.float32)


def kernel(node_features_0, edge_features_0, coords, edge_index, Wq, Wk, Wv,
           Wo, W_out, W_c):
    f32 = jnp.float32
    h = node_features_0[..., 0]
    e = edge_features_0[..., 0]
    src = edge_index[0]
    dst = edge_index[1]
    crp = jnp.pad(coords, ((0, 0), (0, 16 - 3)))

    lane = np.arange(C)
    mh = jnp.asarray((lane[:, None] // HD == lane[None, :] // HD), f32)
    m4 = jnp.asarray((lane[:, None] // HD == np.arange(H)[None, :]), f32)
    rep = jnp.asarray((np.arange(H)[:, None] == lane[None, :] // HD), f32)
    wct = jnp.pad(W_c.T, ((0, 0), (0, 16 - NT)))

    ngrid = pl.cdiv(N, BN)
    egrid = pl.cdiv(E, BE)

    tsrc1, tdst1 = pl.pallas_call(
        _prep1_body,
        grid=(ngrid,),
        in_specs=[_rows((BN, C)), _rows((BN, 16)), _full((C, C)), _full((C, C)),
                  _full((C, C))],
        out_specs=[_rows((BN, 80)), _rows((BN, 48))],
        out_shape=[jax.ShapeDtypeStruct((N, 80), f32),
                   jax.ShapeDtypeStruct((N, 48), f32)],
    )(h, crp, Wq[0], Wk[0][:C], Wv[0][:C])

    g1, g2 = _sc_gather(tsrc1, src, tdst1, dst, 80, 48)

    s1, rb = pl.pallas_call(
        _edge1_body,
        grid=(egrid,),
        in_specs=[_rows((BE, 80)), _rows((BE, 48)), _rows((BE, C)),
                  _full((C + NRBF, C)), _full((C + NRBF, C)),
                  _full((C, C)), _full((C, H))],
        out_specs=[_rows((BE, SW)), _rows((BE, NRBF))],
        out_shape=[jax.ShapeDtypeStruct((E, SW), f32),
                   jax.ShapeDtypeStruct((E, NRBF), f32)],
    )(g1, g2, e, Wk[0][C:], Wv[0][C:], mh, m4)

    p1 = _sc_scatter(s1.T, dst)

    pspec = pl.BlockSpec((SW, BN), lambda i: (0, i))

    h2, tsrc2, tdst2 = pl.pallas_call(
        _upd1_body,
        grid=(ngrid,),
        in_specs=[pspec, _rows((BN, C)), _full((H, C)), _full((C, C)),
                  _full((C, C)), _full((C, C)), _full((C, C))],
        out_specs=[_rows((BN, C)), _rows((BN, 2 * C)), _rows((BN, C))],
        out_shape=[jax.ShapeDtypeStruct((N, C), f32),
                   jax.ShapeDtypeStruct((N, 2 * C), f32),
                   jax.ShapeDtypeStruct((N, C), f32)],
    )(p1, h, rep, Wo[0], Wq[1], Wk[1][:C], Wv[1][:C])

    g1b, g2b = _sc_gather(tsrc2, src, tdst2, dst, 64, 32)

    s2 = pl.pallas_call(
        _edge2_body,
        grid=(egrid,),
        in_specs=[_rows((BE, 64)), _rows((BE, C)), _rows((BE, C)),
                  _rows((BE, NRBF)),
                  _full((C + NRBF, C)), _full((C + NRBF, C)),
                  _full((C, C)), _full((C, H))],
        out_specs=_rows((BE, SW)),
        out_shape=jax.ShapeDtypeStruct((E, SW), f32),
    )(g1b, g2b, e, rb, Wk[1][C:], Wv[1][C:], mh, m4)

    p2 = _sc_scatter(s2.T, dst)

    hs0, cs16 = pl.pallas_call(
        _upd2_body,
        grid=(ngrid,),
        in_specs=[pspec, _rows((BN, C)), _full((H, C)), _full((C, C)),
                  _full((C, C)), _full((C, 16))],
        out_specs=[_rows((BN, C)), _rows((BN, 16))],
        out_shape=[jax.ShapeDtypeStruct((N, C), f32),
                   jax.ShapeDtypeStruct((N, 16), f32)],
    )(p2, h2, rep, Wo[1], W_out, wct)

    return (hs0, cs16[:, :NT])


# speedup vs baseline: 1.0690x; 1.0690x over previous
import functools

import jax
import jax.numpy as jnp
import numpy as np
from jax import lax
from jax.experimental import pallas as pl
from jax.experimental.pallas import tpu as pltpu
from jax.experimental.pallas import tpu_sc as plsc

N = 50000
E = 800000
C = 32
H = 4
HD = 8
NRBF = 16
NT = 15

NC = 2
NS = 16
NW = NC * NS
PER_W = E // NW
CB = 128
NCH = PER_W // CB
TB = PER_W - NCH * CB
SW = 36
CBS = 4000

BN = 2048
BE = 4096

_INV_SQRT_HD = float(1.0 / np.sqrt(HD))


def _sc_gather(tsrc, src_idx, tdst, dst_idx, w1, w2):
    mesh = plsc.VectorSubcoreMesh(core_axis_name="c", subcore_axis_name="s")

    @functools.partial(
        pl.kernel,
        mesh=mesh,
        compiler_params=pltpu.CompilerParams(use_tc_tiling_on_sc=False),
        out_type=[
            jax.ShapeDtypeStruct((E, w1), jnp.float32),
            jax.ShapeDtypeStruct((E, w2), jnp.float32),
        ],
        scratch_types=[
            pltpu.VMEM((CB,), jnp.int32), pltpu.VMEM((CB,), jnp.int32),
            pltpu.VMEM((CB, w1), jnp.float32), pltpu.VMEM((CB, w2), jnp.float32),
            pltpu.VMEM((CB,), jnp.int32), pltpu.VMEM((CB,), jnp.int32),
            pltpu.VMEM((CB, w1), jnp.float32), pltpu.VMEM((CB, w2), jnp.float32),
            pltpu.SemaphoreType.DMA, pltpu.SemaphoreType.DMA,
        ],
    )
    def gather2_k(tsrc_hbm, src_hbm, tdst_hbm, dst_hbm, g1_hbm, g2_hbm,
                  i1a, i2a, r1a, r2a, i1b, i2b, r1b, r2b, sema, semb):
        wid = lax.axis_index("s") * NC + lax.axis_index("c")
        base_w = wid * PER_W

        def stage(base, i1, i2, r1, r2, sem):
            pltpu.sync_copy(src_hbm.at[pl.ds(base, CB)], i1)
            pltpu.sync_copy(dst_hbm.at[pl.ds(base, CB)], i2)
            pltpu.async_copy(tsrc_hbm.at[i1], r1, sem)
            pltpu.async_copy(tdst_hbm.at[i2], r2, sem)

        def finish(base, i1, i2, r1, r2, sem):
            pltpu.make_async_copy(tsrc_hbm.at[i1], r1, sem).wait()
            pltpu.make_async_copy(tdst_hbm.at[i2], r2, sem).wait()
            pltpu.sync_copy(r1, g1_hbm.at[pl.ds(base, CB)])
            pltpu.sync_copy(r2, g2_hbm.at[pl.ds(base, CB)])

        def body(j, carry):
            b0 = base_w + (2 * j) * CB
            b1 = b0 + CB
            stage(b0, i1a, i2a, r1a, r2a, sema)
            stage(b1, i1b, i2b, r1b, r2b, semb)
            finish(b0, i1a, i2a, r1a, r2a, sema)
            finish(b1, i1b, i2b, r1b, r2b, semb)
            return carry

        lax.fori_loop(0, NCH // 2, body, 0)

        def chunk(base, nb, i1, r1, i2, r2):
            pltpu.sync_copy(src_hbm.at[pl.ds(base, nb)], i1)
            pltpu.sync_copy(dst_hbm.at[pl.ds(base, nb)], i2)
            pltpu.async_copy(tsrc_hbm.at[i1], r1, sema)
            pltpu.async_copy(tdst_hbm.at[i2], r2, semb)
            pltpu.make_async_copy(tsrc_hbm.at[i1], r1, sema).wait()
            pltpu.make_async_copy(tdst_hbm.at[i2], r2, semb).wait()
            pltpu.sync_copy(r1, g1_hbm.at[pl.ds(base, nb)])
            pltpu.sync_copy(r2, g2_hbm.at[pl.ds(base, nb)])

        if NCH % 2 == 1:
            chunk(base_w + (NCH - 1) * CB, CB, i1a, r1a, i2a, r2a)
        chunk(base_w + NCH * CB, TB, i1a.at[pl.ds(0, TB)],
              r1a.at[pl.ds(0, TB)], i2a.at[pl.ds(0, TB)],
              r2a.at[pl.ds(0, TB)])

    return gather2_k(tsrc, src_idx, tdst, dst_idx)


def _sc_scatter(sT, dst_idx):
    mesh = plsc.VectorSubcoreMesh(core_axis_name="c", subcore_axis_name="s")
    npair = SW // 2

    @functools.partial(
        pl.kernel,
        mesh=mesh,
        compiler_params=pltpu.CompilerParams(use_tc_tiling_on_sc=False,
                                             needs_layout_passes=False),
        out_type=jax.ShapeDtypeStruct((SW, N), jnp.float32),
        scratch_types=[
            pltpu.VMEM((CBS,), jnp.int32), pltpu.VMEM((CBS,), jnp.float32),
            pltpu.VMEM((CBS,), jnp.float32),
            pltpu.VMEM((CBS,), jnp.int32), pltpu.VMEM((CBS,), jnp.float32),
            pltpu.VMEM((CBS,), jnp.float32),
            pltpu.VMEM((2 * N,), jnp.float32),
            pltpu.SemaphoreType.DMA, pltpu.SemaphoreType.DMA,
        ],
    )
    def scatter_k(s_hbm, dst_hbm, out_hbm, ia, va0, va1, ib, vb0, vb1, acc,
                  sema, semb):
        wid = lax.axis_index("s") * NC + lax.axis_index("c")

        @pl.when(wid < npair)
        def _():
            ch0 = wid * 2
            zv = jnp.zeros((16,), jnp.float32)
            nch2 = E // CBS // 2

            def zb(j, carry):
                acc[pl.ds(j * 16, 16)] = zv
                return carry

            lax.fori_loop(0, (2 * N) // 16, zb, 0)

            def start(i, iv, v0, v1, sem):
                base = i * CBS
                pltpu.async_copy(dst_hbm.at[pl.ds(base, CBS)], iv, sem)
                pltpu.async_copy(s_hbm.at[ch0, pl.ds(base, CBS)], v0, sem)
                pltpu.async_copy(s_hbm.at[ch0 + 1, pl.ds(base, CBS)], v1, sem)

            def drain(i, iv, v0, v1, sem):
                base = i * CBS
                pltpu.make_async_copy(dst_hbm.at[pl.ds(base, CBS)], iv, sem).wait()
                pltpu.make_async_copy(s_hbm.at[ch0, pl.ds(base, CBS)], v0, sem).wait()
                pltpu.make_async_copy(s_hbm.at[ch0 + 1, pl.ds(base, CBS)], v1, sem).wait()

            def proc(iv, v0, v1):
                def inner(k, c2):
                    idx16 = iv[pl.ds(k * 16, 16)]
                    plsc.addupdate_scatter(acc, [idx16], v0[pl.ds(k * 16, 16)])
                    plsc.addupdate_scatter(acc, [idx16 + N], v1[pl.ds(k * 16, 16)])
                    return c2

                lax.fori_loop(0, CBS // 16, inner, 0, unroll=4)

            start(0, ia, va0, va1, sema)

            def body(jj, carry):
                i0 = 2 * jj
                start(i0 + 1, ib, vb0, vb1, semb)
                drain(i0, ia, va0, va1, sema)
                proc(ia, va0, va1)

                @pl.when(i0 + 2 < 2 * nch2)
                def _start_next():
                    start(i0 + 2, ia, va0, va1, sema)

                drain(i0 + 1, ib, vb0, vb1, semb)
                proc(ib, vb0, vb1)
                return carry

            lax.fori_loop(0, nch2, body, 0)
            pltpu.sync_copy(acc.at[pl.ds(0, N)], out_hbm.at[ch0])
            pltpu.sync_copy(acc.at[pl.ds(N, N)], out_hbm.at[ch0 + 1])

    return scatter_k(sT, dst_idx)


def _full(shape):
    return pl.BlockSpec(shape, lambda i: tuple(0 for _ in shape))


def _rows(block_shape):
    return pl.BlockSpec(block_shape, lambda i: (i,) + tuple(0 for _ in block_shape[1:]))


def _prep1_body(h_ref, crp_ref, wq_ref, wkh_ref, wvh_ref, tsrc_ref, tdst_ref):
    h = h_ref[...]
    crp = crp_ref[...]
    q = jnp.dot(h, wq_ref[...], preferred_element_type=jnp.float32)
    ak = jnp.dot(h, wkh_ref[...], preferred_element_type=jnp.float32)
    av = jnp.dot(h, wvh_ref[...], preferred_element_type=jnp.float32)
    tsrc_ref[...] = jnp.concatenate([ak, av, crp], axis=1)
    tdst_ref[...] = jnp.concatenate([q, crp], axis=1)


def _edge1_body(g1_ref, g2_ref, e_ref, wk_ref, wv_ref, mh_ref, m4_ref,
                s_ref, rb_ref):
    g1 = g1_ref[...]
    g2 = g2_ref[...]
    relp = g2[:, C:C + 16] - g1[:, 2 * C:2 * C + 16]
    d = jnp.sqrt(jnp.sum(relp * relp, axis=1, keepdims=True) + 1e-8)
    centers = lax.broadcasted_iota(jnp.int32, (1, NRBF), 1).astype(
        jnp.float32) * (4.0 / (NRBF - 1))
    rb = jnp.exp(-((d - centers) ** 2) * 2.0)
    kin = jnp.concatenate([e_ref[...], rb], axis=1)
    k = g1[:, 0:C] + jnp.dot(kin, wk_ref[...], preferred_element_type=jnp.float32)
    v = g1[:, C:2 * C] + jnp.dot(kin, wv_ref[...], preferred_element_type=jnp.float32)
    prod = g2[:, 0:C] * k
    l32 = jnp.dot(prod, mh_ref[...], preferred_element_type=jnp.float32) * _INV_SQRT_HD
    l4 = jnp.dot(prod, m4_ref[...], preferred_element_type=jnp.float32) * _INV_SQRT_HD
    w32 = jnp.exp(l32)
    w4 = jnp.exp(l4)
    s_ref[...] = jnp.concatenate([w32 * v, w4], axis=1).T
    rb_ref[...] = rb


def _edge2_body(g1_ref, g2_ref, e_ref, rb_ref, wk_ref, wv_ref, mh_ref, m4_ref,
                s_ref):
    g1 = g1_ref[...]
    g2 = g2_ref[...]
    kin = jnp.concatenate([e_ref[...], rb_ref[...]], axis=1)
    k = g1[:, 0:C] + jnp.dot(kin, wk_ref[...], preferred_element_type=jnp.float32)
    v = g1[:, C:2 * C] + jnp.dot(kin, wv_ref[...], preferred_element_type=jnp.float32)
    prod = g2[:, 0:C] * k
    l32 = jnp.dot(prod, mh_ref[...], preferred_element_type=jnp.float32) * _INV_SQRT_HD
    l4 = jnp.dot(prod, m4_ref[...], preferred_element_type=jnp.float32) * _INV_SQRT_HD
    w32 = jnp.exp(l32)
    w4 = jnp.exp(l4)
    s_ref[...] = jnp.concatenate([w32 * v, w4], axis=1).T


def _agg_h(pT, h, rep, wo):
    numerT = pT[0:C, :]
    denT = pT[C:C + H, :]
    den32T = lax.dot_general(rep, denT, (((0,), (0,)), ((), ())),
                             preferred_element_type=jnp.float32)
    aggT = numerT / (den32T + 1e-9)
    upd = lax.dot_general(aggT, wo, (((0,), (0,)), ((), ())),
                          preferred_element_type=jnp.float32)
    return h + jnp.maximum(upd, 0.0)


def _upd1_body(pT_ref, h_ref, rep_ref, wo_ref, wq_ref, wkh_ref, wvh_ref,
               h2_ref, tsrc_ref, tdst_ref):
    h2 = _agg_h(pT_ref[...], h_ref[...], rep_ref[...], wo_ref[...])
    h2_ref[...] = h2
    ak = jnp.dot(h2, wkh_ref[...], preferred_element_type=jnp.float32)
    av = jnp.dot(h2, wvh_ref[...], preferred_element_type=jnp.float32)
    tsrc_ref[...] = jnp.concatenate([ak, av], axis=1)
    tdst_ref[...] = jnp.dot(h2, wq_ref[...], preferred_element_type=jnp.float32)


def _upd2_body(pT_ref, h_ref, rep_ref, wo_ref, wout_ref, wct_ref,
               hs0_ref, cs_ref):
    h2 = _agg_h(pT_ref[...], h_ref[...], rep_ref[...], wo_ref[...])
    hs0 = jnp.dot(h2, wout_ref[...], preferred_element_type=jnp.float32)
    hs0_ref[...] = hs0
    cs_ref[...] = jnp.dot(hs0, wct_ref[...], preferred_element_type=jnp.float32)


def kernel(node_features_0, edge_features_0, coords, edge_index, Wq, Wk, Wv,
           Wo, W_out, W_c):
    f32 = jnp.float32
    h = node_features_0[..., 0]
    e = edge_features_0[..., 0]
    src = edge_index[0]
    dst = edge_index[1]
    crp = jnp.pad(coords, ((0, 0), (0, 16 - 3)))

    lane = np.arange(C)
    mh = jnp.asarray((lane[:, None] // HD == lane[None, :] // HD), f32)
    m4 = jnp.asarray((lane[:, None] // HD == np.arange(H)[None, :]), f32)
    rep = jnp.asarray((np.arange(H)[:, None] == lane[None, :] // HD), f32)
    wct = jnp.pad(W_c.T, ((0, 0), (0, 16 - NT)))

    ngrid = pl.cdiv(N, BN)
    egrid = pl.cdiv(E, BE)

    tsrc1, tdst1 = pl.pallas_call(
        _prep1_body,
        grid=(ngrid,),
        in_specs=[_rows((BN, C)), _rows((BN, 16)), _full((C, C)), _full((C, C)),
                  _full((C, C))],
        out_specs=[_rows((BN, 80)), _rows((BN, 48))],
        out_shape=[jax.ShapeDtypeStruct((N, 80), f32),
                   jax.ShapeDtypeStruct((N, 48), f32)],
    )(h, crp, Wq[0], Wk[0][:C], Wv[0][:C])

    g1, g2 = _sc_gather(tsrc1, src, tdst1, dst, 80, 48)

    s1, rb = pl.pallas_call(
        _edge1_body,
        grid=(egrid,),
        in_specs=[_rows((BE, 80)), _rows((BE, 48)), _rows((BE, C)),
                  _full((C + NRBF, C)), _full((C + NRBF, C)),
                  _full((C, C)), _full((C, H))],
        out_specs=[pl.BlockSpec((SW, BE), lambda i: (0, i)), _rows((BE, NRBF))],
        out_shape=[jax.ShapeDtypeStruct((SW, E), f32),
                   jax.ShapeDtypeStruct((E, NRBF), f32)],
    )(g1, g2, e, Wk[0][C:], Wv[0][C:], mh, m4)

    p1 = _sc_scatter(s1, dst)

    pspec = pl.BlockSpec((SW, BN), lambda i: (0, i))

    h2, tsrc2, tdst2 = pl.pallas_call(
        _upd1_body,
        grid=(ngrid,),
        in_specs=[pspec, _rows((BN, C)), _full((H, C)), _full((C, C)),
                  _full((C, C)), _full((C, C)), _full((C, C))],
        out_specs=[_rows((BN, C)), _rows((BN, 2 * C)), _rows((BN, C))],
        out_shape=[jax.ShapeDtypeStruct((N, C), f32),
                   jax.ShapeDtypeStruct((N, 2 * C), f32),
                   jax.ShapeDtypeStruct((N, C), f32)],
    )(p1, h, rep, Wo[0], Wq[1], Wk[1][:C], Wv[1][:C])

    g1b, g2b = _sc_gather(tsrc2, src, tdst2, dst, 64, 32)

    s2 = pl.pallas_call(
        _edge2_body,
        grid=(egrid,),
        in_specs=[_rows((BE, 64)), _rows((BE, C)), _rows((BE, C)),
                  _rows((BE, NRBF)),
                  _full((C + NRBF, C)), _full((C + NRBF, C)),
                  _full((C, C)), _full((C, H))],
        out_specs=pl.BlockSpec((SW, BE), lambda i: (0, i)),
        out_shape=jax.ShapeDtypeStruct((SW, E), f32),
    )(g1b, g2b, e, rb, Wk[1][C:], Wv[1][C:], mh, m4)

    p2 = _sc_scatter(s2, dst)

    hs0, cs16 = pl.pallas_call(
        _upd2_body,
        grid=(ngrid,),
        in_specs=[pspec, _rows((BN, C)), _full((H, C)), _full((C, C)),
                  _full((C, C)), _full((C, 16))],
        out_specs=[_rows((BN, C)), _rows((BN, 16))],
        out_shape=[jax.ShapeDtypeStruct((N, C), f32),
                   jax.ShapeDtypeStruct((N, 16), f32)],
    )(p2, h2, rep, Wo[1], W_out, wct)

    return (hs0, cs16[:, :NT])
